# baseline (speedup 1.0000x reference)
import functools

import jax
import jax.numpy as jnp
from jax import lax
from jax.experimental import pallas as pl
from jax.experimental.pallas import tpu as pltpu
from jax.experimental.pallas import tpu_sc as plsc

B = 16384
L = 50
D = 64
LANES = 16
NC, NS = 2, 16
NW = NC * NS
ITEMS_PER_W = B // NW
SB = 16
CHUNKS = ITEMS_PER_W // SB
HALF = CHUNKS // 2
ROWS = SB * L
LG = (L + LANES - 1) // LANES
NK = D // LANES


def _glove_body(center_hbm, ctx_hbm, ctxW_hbm, cenW_hbm, ctxb_hbm, cenb_hbm,
                out_hbm,
                bufs, partials_v, out_v, sems):
    wid = lax.axis_index("s") * NC + lax.axis_index("c")
    iota16 = lax.iota(jnp.int32, LANES)

    def row_slices():
        off, res = 0, []
        while off < ROWS:
            n = min(128, ROWS - off)
            res.append((off, n))
            off += n
        return res

    def issue(cbase, buf, sem):
        cidx_v, ctxidx_v, crows_v, ctxrows_v, bias_v, cb_v = buf
        pltpu.sync_copy(center_hbm.at[pl.ds(cbase, SB)], cidx_v)
        pltpu.sync_copy(ctx_hbm.at[pl.ds(cbase * L, ROWS)], ctxidx_v)
        pltpu.async_copy(cenW_hbm.at[cidx_v], crows_v, sem)
        pltpu.async_copy(cenb_hbm.at[cidx_v], cb_v, sem)
        for off, n in row_slices():
            idx = ctxidx_v.at[pl.ds(off, n)]
            pltpu.async_copy(ctxW_hbm.at[idx], ctxrows_v.at[pl.ds(off, n)], sem)
            pltpu.async_copy(ctxb_hbm.at[idx], bias_v.at[pl.ds(off, n)], sem)

    def drain(buf, sem):
        cidx_v, ctxidx_v, crows_v, ctxrows_v, bias_v, cb_v = buf
        pltpu.make_async_copy(cenW_hbm.at[pl.ds(0, SB)], crows_v, sem).wait()
        pltpu.make_async_copy(cenb_hbm.at[pl.ds(0, SB)], cb_v, sem).wait()
        for off, n in row_slices():
            pltpu.make_async_copy(
                ctxW_hbm.at[pl.ds(0, n)], ctxrows_v.at[pl.ds(off, n)],
                sem).wait()
            pltpu.make_async_copy(
                ctxb_hbm.at[pl.ds(0, n)], bias_v.at[pl.ds(off, n)],
                sem).wait()

    def compute(cbase, buf):
        cidx_v, ctxidx_v, crows_v, ctxrows_v, bias_v, cb_v = buf

        def item_body(i, carry2):
            cvecs = []
            for k in range(NK // 2):
                ce, co = plsc.unpack(crows_v[i, pl.ds(k * 2 * LANES, 2 * LANES)],
                                     format=plsc.PackFormat.INTERLEAVED)
                cvecs += [ce, co]
            cb = plsc.load_gather(cb_v, [jnp.full((LANES,), i, jnp.int32)])
            row0 = i * L
            last = row0 + (L - 1)

            def row_body(l, carry3):
                r = row0 + l
                rv = []
                for k in range(NK // 2):
                    re, ro = plsc.unpack(
                        ctxrows_v[r, pl.ds(k * 2 * LANES, 2 * LANES)],
                        format=plsc.PackFormat.INTERLEAVED)
                    rv += [re, ro]
                p = ((rv[0] * cvecs[0] + rv[1] * cvecs[1])
                     + (rv[2] * cvecs[2] + rv[3] * cvecs[3]))
                plsc.store_scatter(partials_v, [r * 17 + iota16], p)
                return carry3

            lax.fori_loop(0, L, row_body, 0, unroll=5)

            for lg in range(LG):
                raw = row0 + lg * LANES + iota16
                lidx = jnp.minimum(raw, last)
                pidx = lidx * 17
                accs = [plsc.load_gather(bias_v, [lidx]) + cb,
                        jnp.zeros((LANES,), jnp.float32),
                        jnp.zeros((LANES,), jnp.float32),
                        jnp.zeros((LANES,), jnp.float32)]
                for k in range(LANES):
                    accs[k % 4] = accs[k % 4] + plsc.load_gather(
                        partials_v, [pidx + k])
                acc = (accs[0] + accs[1]) + (accs[2] + accs[3])
                if (lg + 1) * LANES <= L:
                    plsc.store_scatter(out_v, [lidx], acc)
                else:
                    plsc.store_scatter(out_v, [lidx], acc, mask=raw <= last)
            return carry2

        lax.fori_loop(0, SB, item_body, 0)
        pltpu.sync_copy(out_v.at[pl.ds(0, ROWS)],
                        out_hbm.at[pl.ds(cbase * L, ROWS)])

    buf_a = bufs[:6]
    buf_b = bufs[6:]
    sem_a, sem_b = sems
    wbase = wid * ITEMS_PER_W
    issue(wbase, buf_a, sem_a)

    def pair_body(t, carry):
        cbase_a = wbase + (2 * t) * SB
        drain(buf_a, sem_a)
        issue(cbase_a + SB, buf_b, sem_b)
        compute(cbase_a, buf_a)

        @pl.when(t < HALF - 1)
        def _():
            issue(cbase_a + 2 * SB, buf_a, sem_a)

        drain(buf_b, sem_b)
        compute(cbase_a + SB, buf_b)
        return carry

    lax.fori_loop(0, HALF, pair_body, 0)


def _buf_set():
    return [
        pltpu.VMEM((SB,), jnp.int32),
        pltpu.VMEM((ROWS,), jnp.int32),
        pltpu.VMEM((SB, D), jnp.bfloat16),
        pltpu.VMEM((ROWS, D), jnp.bfloat16),
        pltpu.VMEM((ROWS,), jnp.float32),
        pltpu.VMEM((SB,), jnp.float32),
    ]


def _body_wrap(center_hbm, ctx_hbm, ctxW_hbm, cenW_hbm, ctxb_hbm, cenb_hbm,
               out_hbm,
               a0, a1, a2, a3, a4, a5, b0, b1, b2, b3, b4, b5,
               partials_v, out_v, sem_a, sem_b):
    _glove_body(center_hbm, ctx_hbm, ctxW_hbm, cenW_hbm, ctxb_hbm, cenb_hbm,
                out_hbm,
                [a0, a1, a2, a3, a4, a5, b0, b1, b2, b3, b4, b5],
                partials_v, out_v, (sem_a, sem_b))


_glove_sc = functools.partial(
    pl.kernel,
    out_type=jax.ShapeDtypeStruct((B * L,), jnp.float32),
    mesh=plsc.VectorSubcoreMesh(core_axis_name="c", subcore_axis_name="s"),
    compiler_params=pltpu.CompilerParams(
        needs_layout_passes=False, use_tc_tiling_on_sc=False),
    scratch_types=(
        _buf_set() + _buf_set() + [
            pltpu.VMEM((ROWS * 17 + LANES,), jnp.float32),
            pltpu.VMEM((ROWS + LANES,), jnp.float32),
            pltpu.SemaphoreType.DMA,
            pltpu.SemaphoreType.DMA,
        ]
    ),
)(_body_wrap)


def kernel(center, all_contexts, context_W, center_W, context_b, center_b):
    out = _glove_sc(
        center.reshape(B).astype(jnp.int32),
        all_contexts.reshape(B * L).astype(jnp.int32),
        context_W.astype(jnp.bfloat16),
        center_W.astype(jnp.bfloat16),
        context_b.reshape(-1),
        center_b.reshape(-1),
    )
    return out.reshape(B, L)


# speedup vs baseline: 1.4197x; 1.4197x over previous
import functools

import jax
import jax.numpy as jnp
from jax import lax
from jax.experimental import pallas as pl
from jax.experimental.pallas import tpu as pltpu
from jax.experimental.pallas import tpu_sc as plsc

B = 16384
L = 50
D = 64
W = 2 * D
LANES = 16
NC, NS = 2, 16
NW = NC * NS
ITEMS_PER_W = B // NW
SB = 8
CHUNKS = ITEMS_PER_W // SB
HALF = CHUNKS // 2
ROWS = SB * L
LG = (L + LANES - 1) // LANES
NK = D // LANES


def _glove_body(center_hbm, ctx_hbm, tab_hbm, ctxb_hbm, cenb_hbm,
                out_hbm,
                a0, a1, a2, a3, a4, a5, b0, b1, b2, b3, b4, b5,
                partials_v, out_v, sem_a, sem_b):
    buf_a = (a0, a1, a2, a3, a4, a5)
    buf_b = (b0, b1, b2, b3, b4, b5)
    wid = lax.axis_index("s") * NC + lax.axis_index("c")
    iota16 = lax.iota(jnp.int32, LANES)

    def row_slices():
        off, res = 0, []
        while off < ROWS:
            n = min(128, ROWS - off)
            res.append((off, n))
            off += n
        return res

    def issue(cbase, buf, sem):
        cidx_v, ctxidx_v, crows_v, ctxrows_v, bias_v, cb_v = buf
        pltpu.sync_copy(center_hbm.at[pl.ds(cbase, SB)], cidx_v)
        pltpu.sync_copy(ctx_hbm.at[pl.ds(cbase * L, ROWS)], ctxidx_v)
        pltpu.async_copy(tab_hbm.at[cidx_v], crows_v, sem)
        pltpu.async_copy(cenb_hbm.at[cidx_v], cb_v, sem)
        for off, n in row_slices():
            idx = ctxidx_v.at[pl.ds(off, n)]
            pltpu.async_copy(tab_hbm.at[idx], ctxrows_v.at[pl.ds(off, n)], sem)
            pltpu.async_copy(ctxb_hbm.at[idx], bias_v.at[pl.ds(off, n)], sem)

    def drain(buf, sem):
        cidx_v, ctxidx_v, crows_v, ctxrows_v, bias_v, cb_v = buf
        pltpu.make_async_copy(tab_hbm.at[pl.ds(0, SB)], crows_v, sem).wait()
        pltpu.make_async_copy(cenb_hbm.at[pl.ds(0, SB)], cb_v, sem).wait()
        for off, n in row_slices():
            pltpu.make_async_copy(
                tab_hbm.at[pl.ds(0, n)], ctxrows_v.at[pl.ds(off, n)],
                sem).wait()
            pltpu.make_async_copy(
                ctxb_hbm.at[pl.ds(0, n)], bias_v.at[pl.ds(off, n)],
                sem).wait()

    def compute(cbase, buf):
        cidx_v, ctxidx_v, crows_v, ctxrows_v, bias_v, cb_v = buf

        def item_body(i, carry2):
            cvecs = [crows_v[i, pl.ds(D + k * LANES, LANES)]
                     for k in range(NK)]
            cb = plsc.load_gather(cb_v, [jnp.full((LANES,), i, jnp.int32)])
            row0 = i * L
            last = row0 + (L - 1)

            def row_body(l, carry3):
                r = row0 + l
                rv = [ctxrows_v[r, pl.ds(k * LANES, LANES)] for k in range(NK)]
                p = ((rv[0] * cvecs[0] + rv[1] * cvecs[1])
                     + (rv[2] * cvecs[2] + rv[3] * cvecs[3]))
                plsc.store_scatter(partials_v, [r * 17 + iota16], p)
                return carry3

            lax.fori_loop(0, L, row_body, 0, unroll=5)

            for lg in range(LG):
                raw = row0 + lg * LANES + iota16
                lidx = jnp.minimum(raw, last)
                pidx = lidx * 17
                accs = [plsc.load_gather(bias_v, [lidx]) + cb,
                        jnp.zeros((LANES,), jnp.float32),
                        jnp.zeros((LANES,), jnp.float32),
                        jnp.zeros((LANES,), jnp.float32)]
                for k in range(LANES):
                    accs[k % 4] = accs[k % 4] + plsc.load_gather(
                        partials_v, [pidx + k])
                acc = (accs[0] + accs[1]) + (accs[2] + accs[3])
                if (lg + 1) * LANES <= L:
                    plsc.store_scatter(out_v, [lidx], acc)
                else:
                    plsc.store_scatter(out_v, [lidx], acc, mask=raw <= last)
            return carry2

        lax.fori_loop(0, SB, item_body, 0)
        pltpu.sync_copy(out_v.at[pl.ds(0, ROWS)],
                        out_hbm.at[pl.ds(cbase * L, ROWS)])

    wbase = wid * ITEMS_PER_W
    issue(wbase, buf_a, sem_a)

    def pair_body(t, carry):
        cbase_a = wbase + (2 * t) * SB
        drain(buf_a, sem_a)
        issue(cbase_a + SB, buf_b, sem_b)
        compute(cbase_a, buf_a)

        @pl.when(t < HALF - 1)
        def _():
            issue(cbase_a + 2 * SB, buf_a, sem_a)

        drain(buf_b, sem_b)
        compute(cbase_a + SB, buf_b)
        return carry

    lax.fori_loop(0, HALF, pair_body, 0)


def _buf_set():
    return [
        pltpu.VMEM((SB,), jnp.int32),
        pltpu.VMEM((ROWS,), jnp.int32),
        pltpu.VMEM((SB, W), jnp.float32),
        pltpu.VMEM((ROWS, W), jnp.float32),
        pltpu.VMEM((ROWS,), jnp.float32),
        pltpu.VMEM((SB,), jnp.float32),
    ]


_glove_sc = functools.partial(
    pl.kernel,
    out_type=jax.ShapeDtypeStruct((B * L,), jnp.float32),
    mesh=plsc.VectorSubcoreMesh(core_axis_name="c", subcore_axis_name="s"),
    compiler_params=pltpu.CompilerParams(
        needs_layout_passes=False, use_tc_tiling_on_sc=True),
    scratch_types=(
        _buf_set() + _buf_set() + [
            pltpu.VMEM((ROWS * 17 + LANES,), jnp.float32),
            pltpu.VMEM((ROWS + LANES,), jnp.float32),
            pltpu.SemaphoreType.DMA,
            pltpu.SemaphoreType.DMA,
        ]
    ),
)(_glove_body)


def kernel(center, all_contexts, context_W, center_W, context_b, center_b):
    fused = jnp.concatenate([context_W, center_W], axis=1)
    out = _glove_sc(
        center.reshape(B).astype(jnp.int32),
        all_contexts.reshape(B * L).astype(jnp.int32),
        fused,
        context_b.reshape(-1),
        center_b.reshape(-1),
    )
    return out.reshape(B, L)
